# baseline (speedup 1.0000x reference)
import jax
import jax.numpy as jnp
from jax import lax
from jax.experimental import pallas as pl

_TS = 256


_TB = 1024


def _matvec_kernel(inp_ref, mask_ref, out_ref):
    i = pl.program_id(1)
    v = inp_ref[0, 0]
    out_ref[0, 0, pl.ds(i * _TB, _TB)] = lax.dot_general(
        mask_ref[0],
        v,
        dimension_numbers=(((1,), (0,)), ((), ())),
        preferred_element_type=jnp.float32,
    )


def kernel(input, attention_mask):
    B, S = input.shape
    out = pl.pallas_call(
        _matvec_kernel,
        grid=(B, S // _TB),
        in_specs=[
            pl.BlockSpec((1, 1, S), lambda b, i: (b, 0, 0)),
            pl.BlockSpec((1, _TB, S), lambda b, i: (b, i, 0)),
        ],
        out_specs=pl.BlockSpec((1, 1, S), lambda b, i: (b, 0, 0)),
        out_shape=jax.ShapeDtypeStruct((B, 1, S), jnp.float32),
    )(input.reshape(B, 1, S), attention_mask)
    return out.reshape(B, S)


# speedup vs baseline: 1.0890x; 1.0890x over previous
import jax
import jax.numpy as jnp
from jax import lax
from jax.experimental import pallas as pl

_TS = 256


def _matvec_kernel(inp_ref, mask_ref, out_ref):
    i = pl.program_id(0)
    out_ref[:, pl.ds(i * _TS, _TS)] = lax.dot_general(
        mask_ref[...],
        inp_ref[...],
        dimension_numbers=(((2,), (1,)), ((0,), (0,))),
        preferred_element_type=jnp.float32,
    )


def kernel(input, attention_mask):
    B, S = input.shape
    return pl.pallas_call(
        _matvec_kernel,
        grid=(S // _TS,),
        in_specs=[
            pl.BlockSpec((B, S), lambda i: (0, 0)),
            pl.BlockSpec((B, _TS, S), lambda i: (0, i, 0)),
        ],
        out_specs=pl.BlockSpec((B, S), lambda i: (0, 0)),
        out_shape=jax.ShapeDtypeStruct((B, S), jnp.float32),
    )(input, attention_mask)
